# Initial kernel scaffold; baseline (speedup 1.0000x reference)
import jax
import jax.numpy as jnp
from jax.experimental import pallas as pl


def kernel(query_xyz, support_xyz, query_mask, support_mask, queryandkey, value, attention_centrality):
    raise NotImplementedError("write your pallas kernel here")


# speedup vs baseline: 1.1585x; 1.1585x over previous
import functools

import jax
import jax.numpy as jnp
from jax.experimental import pallas as pl
from jax.experimental.pallas import tpu as pltpu

_RADIUS = 0.2
_NS = 16
_BIG = 1e10


def _ballq_body(q_ref, s_ref, idx_ref, mask_ref):
    n, m = q_ref.shape[0], s_ref.shape[1]
    d2 = None
    for c in range(3):
        qc = q_ref[:, c : c + 1]
        sc = s_ref[c : c + 1, :]
        diff = qc - sc
        sq = diff * diff
        d2 = sq if d2 is None else d2 + sq
    lanes = jax.lax.broadcasted_iota(jnp.int32, (n, m), 1)
    r2 = jnp.float32(_RADIUS * _RADIUS)
    idx0 = None
    for k in range(_NS):
        mv = jnp.min(d2, axis=1, keepdims=True)
        cand = jnp.where(d2 == mv, lanes, m)
        amin = jnp.min(cand, axis=1, keepdims=True)
        within = mv <= r2
        if k == 0:
            idx0 = amin
        idx_ref[:, k : k + 1] = jnp.where(within, amin, idx0)
        mask_ref[:, k : k + 1] = within.astype(jnp.int32)
        d2 = jnp.where(lanes == amin, jnp.float32(_BIG), d2)


def _ballq_wrap(body):
    def f(q_ref, s_ref, idx_ref, mask_ref):
        body(q_ref.at[0], s_ref.at[0], idx_ref.at[0], mask_ref.at[0])
    return f


def _actopk_body(a_ref, idx_ref):
    r, m = a_ref.shape
    a = a_ref[...]
    lanes = jax.lax.broadcasted_iota(jnp.int32, (r, m), 1)
    for k in range(_NS):
        mv = jnp.max(a, axis=1, keepdims=True)
        cand = jnp.where(a == mv, lanes, m)
        amin = jnp.min(cand, axis=1, keepdims=True)
        idx_ref[:, k : k + 1] = amin
        a = jnp.where(lanes == amin, jnp.float32(-_BIG), a)


def _ac_topk(ac_flat):
    r, m = ac_flat.shape
    return pl.pallas_call(
        _actopk_body,
        out_shape=jax.ShapeDtypeStruct((r, _NS), jnp.int32),
    )(ac_flat)


def kernel(query_xyz, support_xyz, query_mask, support_mask, queryandkey, value, attention_centrality):
    b, n, _ = query_xyz.shape
    m = support_xyz.shape[1]
    groups = attention_centrality.shape[1]
    dim_value = value.shape[1] // 4

    support_t = jnp.transpose(support_xyz, (0, 2, 1))
    query_t = jnp.transpose(query_xyz, (0, 2, 1))

    idx, maski = _ball_query_call(query_xyz, support_t)
    idx_mask = maski.astype(bool)

    localvalue = value[:, dim_value:, :]
    nonlocalvalue = value[:, :dim_value, :]

    def group(f, i):
        return jax.vmap(lambda ff, ii: ff[:, ii])(f, i)

    q_e = query_t[..., None]
    g_xyz = (group(support_t, idx) - q_e) / _RADIUS
    g_full = group(jnp.concatenate([localvalue, queryandkey], axis=1), idx)
    dl = localvalue.shape[1]
    new_localfeatures = jnp.concatenate([g_xyz, g_full[:, :dl]], axis=1)
    queryandkey_out = g_full[:, dl:, :, 0]

    idx_ac = _ac_topk(attention_centrality.reshape(b * groups, m)).reshape(b, groups, _NS)
    parts = []
    for j in range(groups):
        idx_j = jnp.broadcast_to(idx_ac[:, j, None, :], (b, n, _NS))
        gx = group(support_t, idx_j) - q_e
        gv = group(nonlocalvalue, idx_j)
        parts.append(jnp.concatenate([gx, gv], axis=1)[:, None])
    new_nonlocalfeatures = jnp.concatenate(parts, axis=1)

    return (new_localfeatures, new_nonlocalfeatures, idx_mask, queryandkey_out)


def _ball_query_call(query_xyz, support_t):
    b, n, _ = query_xyz.shape
    m = support_t.shape[2]
    return pl.pallas_call(
        _ballq_wrap(_ballq_body),
        grid=(b,),
        in_specs=[
            pl.BlockSpec((1, n, 3), lambda i: (i, 0, 0)),
            pl.BlockSpec((1, 3, m), lambda i: (i, 0, 0)),
        ],
        out_specs=[
            pl.BlockSpec((1, n, _NS), lambda i: (i, 0, 0)),
            pl.BlockSpec((1, n, _NS), lambda i: (i, 0, 0)),
        ],
        out_shape=[
            jax.ShapeDtypeStruct((b, n, _NS), jnp.int32),
            jax.ShapeDtypeStruct((b, n, _NS), jnp.int32),
        ],
    )(query_xyz, support_t)
